# baseline (speedup 1.0000x reference)
import functools

import jax
import jax.numpy as jnp
from jax.experimental import pallas as pl
from jax.experimental.pallas import tpu as pltpu

BAND = 10
STRIP = 128
WIN = 384
KCH = 256


def _band_strips(get_tile, get_src, i, ti, n):
    outs = []
    for k in range(ti // STRIP):
        r0 = i * ti + k * STRIP
        c0 = pl.multiple_of(
            jnp.maximum(0, jnp.minimum(r0 - STRIP, n - WIN)), STRIP)
        a = get_tile(k * STRIP, STRIP, c0, WIN)
        rr = jax.lax.broadcasted_iota(jnp.int32, (STRIP, WIN), 0) + r0
        cc = jax.lax.broadcasted_iota(jnp.int32, (STRIP, WIN), 1) + c0
        am = jnp.where(jnp.abs(rr - cc) <= BAND, a, jnp.bfloat16(0))
        sv = get_src(c0, WIN)
        outs.append(jnp.dot(am, sv, preferred_element_type=jnp.float32))
    return outs


def _gcn_kernel(x_ref, adj_ref, w1_ref, b1_ref, wb1_ref, bb1_ref,
                w34_ref, b3_ref, bb3_ref, wfc_ref, bfc_ref,
                out_ref, abf, hw, hwb,
                *, ti, ni, n, nf, nh1, nh2):
    bidx = pl.program_id(0)
    s = pl.program_id(1)
    i = pl.program_id(2)
    rows = pl.ds(i * ti, ti)

    @pl.when(s == 0)
    def _layer1():
        ax = jnp.zeros((ti, nf), jnp.float32)
        for k in range(n // KCH):
            cols = pl.ds(k * KCH, KCH)
            a16 = adj_ref[0, :, cols].astype(jnp.bfloat16)
            abf[rows, cols] = a16
            ax = ax + jnp.dot(a16, x_ref[0, pl.ds(k * KCH, KCH), :],
                              preferred_element_type=jnp.float32)
        nl = jnp.dot(ax.astype(jnp.bfloat16), w1_ref[...],
                     preferred_element_type=jnp.float32)

        def tile(r0, rlen, c0, clen):
            return adj_ref[0, pl.ds(r0, rlen),
                           pl.ds(c0, clen)].astype(jnp.bfloat16)

        bxs = _band_strips(tile,
                           lambda c0, cl: x_ref[0, pl.ds(c0, cl), :],
                           i, ti, n)
        bx = jnp.concatenate(bxs, axis=0).astype(jnp.bfloat16)
        loc = jnp.dot(bx, wb1_ref[...], preferred_element_type=jnp.float32)
        h = (jax.nn.relu(nl + b1_ref[...])
             + jax.nn.relu(loc + bb1_ref[...]))
        h16 = h.astype(jnp.bfloat16)
        t = jnp.dot(h16, w34_ref[...], preferred_element_type=jnp.float32)
        hw[rows, :] = t[:, 0:nh2].astype(jnp.bfloat16)
        hwb[rows, :] = t[:, nh2:2 * nh2].astype(jnp.bfloat16)

    @pl.when(s == 1)
    def _layer2():
        nl = jnp.dot(abf[rows, :], hw[...], preferred_element_type=jnp.float32)

        def tile(r0, rlen, c0, clen):
            return abf[pl.ds(i * ti + r0, rlen), pl.ds(c0, clen)]

        locs = _band_strips(tile,
                            lambda c0, cl: hwb[pl.ds(c0, cl), :],
                            i, ti, n)
        loc = jnp.concatenate(locs, axis=0)
        h = (jax.nn.relu(nl + b3_ref[...])
             + jax.nn.relu(loc + bb3_ref[...]))
        pooled = jnp.sum(h, axis=0, keepdims=True) * (1.0 / n)
        contrib = jnp.dot(pooled, wfc_ref[...],
                          preferred_element_type=jnp.float32)

        @pl.when(i == 0)
        def _():
            out_ref[pl.ds(bidx, 1), :] = bfc_ref[...] + contrib

        @pl.when(i > 0)
        def _():
            out_ref[pl.ds(bidx, 1), :] += contrib


def kernel(x, adj, W1, b1, Wb1, bb1, W3, b3, Wb3, bb3, Wfc, bfc):
    B, N, NFEAT = x.shape
    NH1 = W1.shape[1]
    NH2 = W3.shape[1]
    NCLASS = Wfc.shape[1]

    TI = min(1024, N)
    NI = N // TI

    b1r = b1.reshape(1, NH1)
    bb1r = bb1.reshape(1, NH1)
    b3r = b3.reshape(1, NH2)
    bb3r = bb3.reshape(1, NH2)
    bfcr = bfc.reshape(1, NCLASS)

    x16 = x.astype(jnp.bfloat16)
    W1c = W1.astype(jnp.bfloat16)
    Wb1c = Wb1.astype(jnp.bfloat16)
    W34c = jnp.concatenate([W3, Wb3], axis=1).astype(jnp.bfloat16)

    out = pl.pallas_call(
        functools.partial(_gcn_kernel, ti=TI, ni=NI, n=N, nf=NFEAT,
                          nh1=NH1, nh2=NH2),
        grid=(B, 2, NI),
        in_specs=[
            pl.BlockSpec((1, N, NFEAT), lambda b, s, i: (b, 0, 0)),
            pl.BlockSpec((1, TI, N),
                         lambda b, s, i, _ni=NI: (
                             b, jnp.where(s == 0, i, _ni - 1), 0)),
            pl.BlockSpec((NFEAT, NH1), lambda b, s, i: (0, 0)),
            pl.BlockSpec((1, NH1), lambda b, s, i: (0, 0)),
            pl.BlockSpec((NFEAT, NH1), lambda b, s, i: (0, 0)),
            pl.BlockSpec((1, NH1), lambda b, s, i: (0, 0)),
            pl.BlockSpec((NH1, 2 * NH2), lambda b, s, i: (0, 0)),
            pl.BlockSpec((1, NH2), lambda b, s, i: (0, 0)),
            pl.BlockSpec((1, NH2), lambda b, s, i: (0, 0)),
            pl.BlockSpec((NH2, NCLASS), lambda b, s, i: (0, 0)),
            pl.BlockSpec((1, NCLASS), lambda b, s, i: (0, 0)),
        ],
        out_specs=pl.BlockSpec((B, NCLASS), lambda b, s, i: (0, 0)),
        out_shape=jax.ShapeDtypeStruct((B, NCLASS), jnp.float32),
        scratch_shapes=[
            pltpu.VMEM((N, N), jnp.bfloat16),
            pltpu.VMEM((N, NH2), jnp.bfloat16),
            pltpu.VMEM((N, NH2), jnp.bfloat16),
        ],
    )(x16, adj, W1c, b1r, Wb1c, bb1r, W34c, b3r, bb3r, Wfc, bfcr)

    return out


# speedup vs baseline: 1.0343x; 1.0343x over previous
import functools

import jax
import jax.numpy as jnp
from jax.experimental import pallas as pl
from jax.experimental.pallas import tpu as pltpu

BAND = 10
STRIP = 128
WIN = 384
KCH = 256


def _band_strips(get_tile, get_src, i, ti, n):
    outs = []
    for k in range(ti // STRIP):
        r0 = i * ti + k * STRIP
        c0 = pl.multiple_of(
            jnp.maximum(0, jnp.minimum(r0 - STRIP, n - WIN)), STRIP)
        a = get_tile(k * STRIP, STRIP, c0, WIN)
        rr = jax.lax.broadcasted_iota(jnp.int32, (STRIP, WIN), 0) + r0
        cc = jax.lax.broadcasted_iota(jnp.int32, (STRIP, WIN), 1) + c0
        am = jnp.where(jnp.abs(rr - cc) <= BAND, a, jnp.bfloat16(0))
        sv = get_src(c0, WIN)
        outs.append(jnp.dot(am, sv, preferred_element_type=jnp.float32))
    return outs


def _gcn_kernel(x_ref, adj_ref, w1_ref, b1_ref, wb1_ref, bb1_ref,
                w3_ref, b3_ref, wb3_ref, bb3_ref, wfc_ref, bfc_ref,
                out_ref, abf, hw, hwb, loc_ref,
                *, ti, ni, n, nf, nh1, nh2):
    bidx = pl.program_id(0)
    s = pl.program_id(1)
    i = pl.program_id(2)
    rows = pl.ds(i * ti, ti)

    @pl.when(s == 0)
    def _layer1():
        ax = jnp.zeros((ti, nf), jnp.float32)
        for k in range(n // KCH):
            cols = pl.ds(k * KCH, KCH)
            a16 = adj_ref[0, :, cols].astype(jnp.bfloat16)
            abf[rows, cols] = a16
            ax = ax + jnp.dot(a16, x_ref[0, pl.ds(k * KCH, KCH), :],
                              preferred_element_type=jnp.float32)
        nl = jnp.dot(ax.astype(jnp.bfloat16), w1_ref[...],
                     preferred_element_type=jnp.float32)

        def tile(r0, rlen, c0, clen):
            return adj_ref[0, pl.ds(r0, rlen),
                           pl.ds(c0, clen)].astype(jnp.bfloat16)

        bxs = _band_strips(tile,
                           lambda c0, cl: x_ref[0, pl.ds(c0, cl), :],
                           i, ti, n)
        for k, bx in enumerate(bxs):
            loc_ref[k * STRIP:(k + 1) * STRIP, :] = jnp.dot(
                bx.astype(jnp.bfloat16), wb1_ref[...],
                preferred_element_type=jnp.float32)
        h = (jax.nn.relu(nl + b1_ref[...])
             + jax.nn.relu(loc_ref[...] + bb1_ref[...]))
        h16 = h.astype(jnp.bfloat16)
        t = jnp.dot(h16, w3_ref[...], preferred_element_type=jnp.float32)
        hw[rows, :] = t.astype(jnp.bfloat16)
        t2 = jnp.dot(h16, wb3_ref[...], preferred_element_type=jnp.float32)
        hwb[rows, :] = t2.astype(jnp.bfloat16)

    @pl.when(s == 1)
    def _layer2():
        nl = jnp.dot(abf[rows, :], hw[...], preferred_element_type=jnp.float32)

        def tile(r0, rlen, c0, clen):
            return abf[pl.ds(i * ti + r0, rlen), pl.ds(c0, clen)]

        locs = _band_strips(tile,
                            lambda c0, cl: hwb[pl.ds(c0, cl), :],
                            i, ti, n)
        loc = jnp.concatenate(locs, axis=0)
        h = (jax.nn.relu(nl + b3_ref[...])
             + jax.nn.relu(loc + bb3_ref[...]))
        pooled = jnp.sum(h, axis=0, keepdims=True) * (1.0 / n)
        contrib = jnp.dot(pooled, wfc_ref[...],
                          preferred_element_type=jnp.float32)

        @pl.when(i == 0)
        def _():
            out_ref[pl.ds(bidx, 1), :] = bfc_ref[...] + contrib

        @pl.when(i > 0)
        def _():
            out_ref[pl.ds(bidx, 1), :] += contrib


def kernel(x, adj, W1, b1, Wb1, bb1, W3, b3, Wb3, bb3, Wfc, bfc):
    B, N, NFEAT = x.shape
    NH1 = W1.shape[1]
    NH2 = W3.shape[1]
    NCLASS = Wfc.shape[1]

    TI = min(1024, N)
    NI = N // TI

    b1r = b1.reshape(1, NH1)
    bb1r = bb1.reshape(1, NH1)
    b3r = b3.reshape(1, NH2)
    bb3r = bb3.reshape(1, NH2)
    bfcr = bfc.reshape(1, NCLASS)

    x16 = x.astype(jnp.bfloat16)
    W1c = W1.astype(jnp.bfloat16)
    Wb1c = Wb1.astype(jnp.bfloat16)
    W3c = W3.astype(jnp.bfloat16)
    Wb3c = Wb3.astype(jnp.bfloat16)

    out = pl.pallas_call(
        functools.partial(_gcn_kernel, ti=TI, ni=NI, n=N, nf=NFEAT,
                          nh1=NH1, nh2=NH2),
        grid=(B, 2, NI),
        in_specs=[
            pl.BlockSpec((1, N, NFEAT), lambda b, s, i: (b, 0, 0)),
            pl.BlockSpec((1, TI, N),
                         lambda b, s, i, _ni=NI: (
                             b, jnp.where(s == 0, i, _ni - 1), 0)),
            pl.BlockSpec((NFEAT, NH1), lambda b, s, i: (0, 0)),
            pl.BlockSpec((1, NH1), lambda b, s, i: (0, 0)),
            pl.BlockSpec((NFEAT, NH1), lambda b, s, i: (0, 0)),
            pl.BlockSpec((1, NH1), lambda b, s, i: (0, 0)),
            pl.BlockSpec((NH1, NH2), lambda b, s, i: (0, 0)),
            pl.BlockSpec((1, NH2), lambda b, s, i: (0, 0)),
            pl.BlockSpec((NH1, NH2), lambda b, s, i: (0, 0)),
            pl.BlockSpec((1, NH2), lambda b, s, i: (0, 0)),
            pl.BlockSpec((NH2, NCLASS), lambda b, s, i: (0, 0)),
            pl.BlockSpec((1, NCLASS), lambda b, s, i: (0, 0)),
        ],
        out_specs=pl.BlockSpec((B, NCLASS), lambda b, s, i: (0, 0)),
        out_shape=jax.ShapeDtypeStruct((B, NCLASS), jnp.float32),
        scratch_shapes=[
            pltpu.VMEM((N, N), jnp.bfloat16),
            pltpu.VMEM((N, NH2), jnp.bfloat16),
            pltpu.VMEM((N, NH2), jnp.bfloat16),
            pltpu.VMEM((TI, NH1), jnp.float32),
        ],
    )(x16, adj, W1c, b1r, Wb1c, bb1r, W3c, b3r, Wb3c, bb3r, Wfc, bfcr)

    return out
